# baseline (speedup 1.0000x reference)
import functools

import jax
import jax.numpy as jnp
from jax.experimental import pallas as pl
from jax.experimental.pallas import tpu as pltpu


_VMEM_LIMIT_BYTES = 48 * 1024 * 1024


def _pool_linear_kernel(x_ref, w_ref, b_ref, o_ref, acc_ref, *,
                        inv_hw, n_label, tb_x, n_s):
    s = pl.program_id(1)
    acc_ref[pl.ds(s * tb_x, tb_x), :] = jnp.sum(x_ref[...], axis=0)

    @pl.when(s == n_s - 1)
    def _finalize():
        pooled = acc_ref[...] * inv_hw
        y = jnp.dot(pooled, w_ref[...], preferred_element_type=jnp.float32)
        y = (y + b_ref[...]).astype(o_ref.dtype)
        o_ref[...] = y.T[:n_label, :]


def kernel(x, weight_t, bias2):
    B, C, H, W = x.shape
    HW = H * W
    Lp = weight_t.shape[1]
    n_label = 1000

    x3 = x.transpose(2, 3, 0, 1).reshape(HW, B, C)

    tb = 256 if B % 256 == 0 and B >= 512 else (
        128 if B % 128 == 0 and B >= 256 else B)
    tb_x = 64 if tb % 64 == 0 and HW * 64 * C * x.dtype.itemsize >= 2**22 else tb
    n_s = tb // tb_x
    grid = (B // tb, n_s)

    cost = pl.CostEstimate(
        flops=int(B * C * HW + 2 * B * C * Lp),
        transcendentals=0,
        bytes_accessed=int(x.dtype.itemsize * B * C * HW
                           + 4 * (C * Lp + Lp + B * n_label)))

    out = pl.pallas_call(
        functools.partial(_pool_linear_kernel, inv_hw=float(1.0 / HW),
                          n_label=n_label, tb_x=tb_x, n_s=n_s),
        out_shape=jax.ShapeDtypeStruct((n_label, B), jnp.float32),
        grid=grid,
        in_specs=[
            pl.BlockSpec((HW, tb_x, C), lambda i, s: (0, i * n_s + s, 0)),
            pl.BlockSpec((C, Lp), lambda i, s: (0, 0)),
            pl.BlockSpec((1, Lp), lambda i, s: (0, 0)),
        ],
        out_specs=pl.BlockSpec((n_label, tb), lambda i, s: (0, i)),
        scratch_shapes=[pltpu.VMEM((tb, C), jnp.float32)],
        compiler_params=pltpu.CompilerParams(
            dimension_semantics=("parallel", "arbitrary"),
            vmem_limit_bytes=_VMEM_LIMIT_BYTES),
        cost_estimate=cost,
    )(x3, weight_t, bias2)

    return out.T


# speedup vs baseline: 1.0140x; 1.0140x over previous
import functools

import jax
import jax.numpy as jnp
from jax.experimental import pallas as pl
from jax.experimental.pallas import tpu as pltpu


_VMEM_LIMIT_BYTES = 48 * 1024 * 1024


def _pool_linear_kernel(x_ref, w_ref, b_ref, o_ref, acc_ref, *,
                        inv_hw, n_label, tb_x, n_s):
    s = pl.program_id(1)
    acc_ref[pl.ds(s * tb_x, tb_x), :] = jnp.sum(x_ref[...], axis=0)

    @pl.when(s == n_s - 1)
    def _finalize():
        pooled = acc_ref[...] * inv_hw
        y = jnp.dot(pooled, w_ref[...], preferred_element_type=jnp.float32)
        y = (y + b_ref[...]).astype(o_ref.dtype)
        o_ref[...] = y.T[:n_label, :]


def kernel(x, weight_t, bias2):
    B, C, H, W = x.shape
    HW = H * W
    Lp = weight_t.shape[1]
    n_label = 1000

    x3 = x.transpose(2, 3, 0, 1).reshape(HW, B, C)

    tb = 128 if B % 128 == 0 and B >= 256 else B
    tb_x = 64 if tb % 64 == 0 and HW * 64 * C * x.dtype.itemsize >= 2**22 else tb
    n_s = tb // tb_x
    grid = (B // tb, n_s)

    cost = pl.CostEstimate(
        flops=int(B * C * HW + 2 * B * C * Lp),
        transcendentals=0,
        bytes_accessed=int(x.dtype.itemsize * B * C * HW
                           + 4 * (C * Lp + Lp + B * n_label)))

    out = pl.pallas_call(
        functools.partial(_pool_linear_kernel, inv_hw=float(1.0 / HW),
                          n_label=n_label, tb_x=tb_x, n_s=n_s),
        out_shape=jax.ShapeDtypeStruct((n_label, B), jnp.float32),
        grid=grid,
        in_specs=[
            pl.BlockSpec((HW, tb_x, C), lambda i, s: (0, i * n_s + s, 0)),
            pl.BlockSpec((C, Lp), lambda i, s: (0, 0)),
            pl.BlockSpec((1, Lp), lambda i, s: (0, 0)),
        ],
        out_specs=pl.BlockSpec((n_label, tb), lambda i, s: (0, i)),
        scratch_shapes=[pltpu.VMEM((tb, C), jnp.float32)],
        compiler_params=pltpu.CompilerParams(
            dimension_semantics=("parallel", "arbitrary"),
            vmem_limit_bytes=_VMEM_LIMIT_BYTES),
        cost_estimate=cost,
    )(x3, weight_t, bias2)

    return out.T
